# baseline (speedup 1.0000x reference)
import functools

import jax
import jax.numpy as jnp
from jax import lax
from jax.experimental import pallas as pl
from jax.experimental.pallas import tpu as pltpu
from jax.experimental.pallas import tpu_sc as plsc

F = 128
RB = 16
CUTOFF = 0.5
N_BLOCKS = 4

E_PAD = 163840
N_PAD = 10240
BE = 2048
BA = 2048
GW = 256
CH = 320

_VMESH = plsc.VectorSubcoreMesh(core_axis_name="c", subcore_axis_name="s")


def _h_body(z_ref, emb_ref, h_ref):
    z = z_ref[...]
    cols = lax.broadcasted_iota(jnp.int32, (BA, F), 1)
    onehot = (cols == z).astype(jnp.float32)
    h_ref[...] = jnp.dot(onehot, emb_ref[...],
                         preferred_element_type=jnp.float32)


def _rbf(d_ref):
    dd = d_ref[...] + 1e-3
    n = lax.broadcasted_iota(jnp.int32, (BE, RB), 1).astype(jnp.float32) + 1.0
    rbf = jnp.sqrt(2.0 / CUTOFF) * jnp.sin(n * jnp.pi * dd / CUTOFF) / dd
    u = dd / CUTOFF
    fcut = 0.5 * (jnp.cos(jnp.pi * u) + 1.0) * (u < 1.0).astype(jnp.float32)
    return rbf * fcut


def _silu(x):
    return x * jax.nn.sigmoid(x)


def _embed_body(hs_ref, hd_ref, d_ref, w1_ref, w2_ref, w3_ref, b_ref, m_ref):
    x = (jnp.dot(hs_ref[...], w1_ref[...], preferred_element_type=jnp.float32)
         + jnp.dot(hd_ref[...], w2_ref[...], preferred_element_type=jnp.float32)
         + jnp.dot(_rbf(d_ref), w3_ref[...], preferred_element_type=jnp.float32)
         + b_ref[...])
    m_ref[...] = _silu(x)


def _msg_first_body(m_ref, d_ref, wr_ref, wm_ref, mm_ref):
    g = jnp.dot(_rbf(d_ref), wr_ref[...], preferred_element_type=jnp.float32)
    x = jnp.dot(m_ref[...] * g, wm_ref[...], preferred_element_type=jnp.float32)
    mm_ref[...] = _silu(x)


def _msg_body(m_ref, agg_ref, d_ref, wr_ref, wm_ref, mm_ref, mnew_ref):
    m = m_ref[...] + agg_ref[...]
    g = jnp.dot(_rbf(d_ref), wr_ref[...], preferred_element_type=jnp.float32)
    x = jnp.dot(m * g, wm_ref[...], preferred_element_type=jnp.float32)
    mm_ref[...] = _silu(x)
    mnew_ref[...] = m


def _add_body(m_ref, agg_ref, o_ref):
    o_ref[...] = m_ref[...] + agg_ref[...]


def _final_body(p0_ref, p1_ref, w16_ref, b_ref, o_ref):
    s = p0_ref[...] + p1_ref[...]
    e = jnp.dot(s, w16_ref[...], preferred_element_type=jnp.float32)
    o_ref[...] = e[:, 0:1] + b_ref[...]


def _full(shape):
    return pl.BlockSpec(shape, lambda i: tuple(0 for _ in shape))


def _sc_gather(tables, idx):
    e = idx.shape[0]
    width = tables[0].shape[1]
    idx2 = idx.reshape(1, e)

    @functools.partial(
        pl.kernel,
        out_type=jax.ShapeDtypeStruct((e, width), tables[0].dtype),
        mesh=_VMESH,
    )
    def k(*refs):
        tab_refs = refs[:len(tables)]
        i_hbm = refs[len(tables)]
        o_hbm = refs[len(tables) + 1]

        def body(i_vmem, o_vmem):
            pltpu.sync_copy(tab_refs[0].at[i_vmem.at[0]], o_vmem)
            for t in tab_refs[1:]:
                pltpu.sync_copy(t.at[i_vmem.at[0]], o_vmem, add=True)

        pltpu.emit_pipeline(
            body,
            grid=(e // GW,),
            in_specs=[pl.BlockSpec((1, GW), lambda i: (0, i))],
            out_specs=[pl.BlockSpec((GW, width), lambda i: (i, 0))],
            core_axis_name=("c", "s"),
            dimension_semantics=(pltpu.PARALLEL,),
        )(i_hbm, o_hbm)

    return k(*tables, idx2)


def _sc_segment_sum(vals, dst3, zeros):
    width = vals.shape[1]
    nch_w = E_PAD // CH // 32
    rs = N_PAD // 16

    @functools.partial(
        pl.kernel,
        out_type=jax.ShapeDtypeStruct((2, N_PAD, width), jnp.float32),
        mesh=_VMESH,
        scratch_types=[
            pltpu.VMEM_SHARED((N_PAD, width), jnp.float32),
            pltpu.VMEM((CH, width), jnp.float32),
            pltpu.VMEM((1, CH), jnp.int32),
        ],
    )
    def k(vals_hbm, dst_hbm, z_hbm, out_hbm, agg_sh, rows_v, idx_v):
        cid = lax.axis_index("c")
        sid = lax.axis_index("s")
        pltpu.sync_copy(z_hbm.at[pl.ds(sid * rs, rs)],
                        agg_sh.at[pl.ds(sid * rs, rs)])
        plsc.subcore_barrier()

        w = cid * 16 + sid

        @pl.loop(0, nch_w)
        def _(j):
            ch = w * nch_w + j
            pltpu.sync_copy(dst_hbm.at[ch], idx_v)
            pltpu.sync_copy(vals_hbm.at[pl.ds(ch * CH, CH)], rows_v)
            pltpu.sync_copy(rows_v, agg_sh.at[idx_v.at[0]], add=True)

        plsc.subcore_barrier()
        pltpu.sync_copy(agg_sh.at[pl.ds(sid * rs, rs)],
                        out_hbm.at[cid].at[pl.ds(sid * rs, rs)])

    return k(vals, dst3, zeros)


def kernel(atomic_numbers, positions, pair_indices, d_ij,
           atomic_subsystem_indices, emb_table, w_embed, b_embed,
           w_rbf, w_msg, w_out, b_out):
    n_atoms = atomic_numbers.shape[0]
    e = pair_indices.shape[1]
    f32 = jnp.float32

    pad_a = N_PAD - n_atoms
    pad_e = E_PAD - e
    z_p = jnp.concatenate(
        [atomic_numbers.astype(jnp.int32),
         jnp.full((pad_a,), 101, jnp.int32)]).reshape(N_PAD, 1)
    emb_pad = jnp.zeros((F, F), f32).at[:emb_table.shape[0]].set(emb_table)
    src_p = jnp.concatenate([pair_indices[0].astype(jnp.int32),
                             jnp.full((pad_e,), n_atoms, jnp.int32)])
    dst_p = jnp.concatenate([pair_indices[1].astype(jnp.int32),
                             jnp.full((pad_e,), n_atoms, jnp.int32)])
    dst3 = dst_p.reshape(E_PAD // CH, 1, CH)
    d_p = jnp.concatenate([d_ij.astype(f32),
                           jnp.zeros((pad_e, 1), f32)])
    w1 = w_embed[:F]
    w2 = w_embed[F:2 * F]
    w3 = w_embed[2 * F:]
    b2 = b_embed.reshape(1, F)
    w16 = jnp.tile(w_out, (1, 16))
    b11 = b_out.reshape(1, 1)
    zeros_f = jnp.zeros((N_PAD, F), f32)

    h = pl.pallas_call(
        _h_body,
        grid=(N_PAD // BA,),
        in_specs=[pl.BlockSpec((BA, 1), lambda i: (i, 0)), _full((F, F))],
        out_specs=pl.BlockSpec((BA, F), lambda i: (i, 0)),
        out_shape=jax.ShapeDtypeStruct((N_PAD, F), f32),
    )(z_p, emb_pad)

    hsrc = _sc_gather([h], src_p)
    hdst = _sc_gather([h], dst_p)

    m = pl.pallas_call(
        _embed_body,
        grid=(E_PAD // BE,),
        in_specs=[pl.BlockSpec((BE, F), lambda i: (i, 0)),
                  pl.BlockSpec((BE, F), lambda i: (i, 0)),
                  pl.BlockSpec((BE, 1), lambda i: (i, 0)),
                  _full((F, F)), _full((F, F)), _full((RB, F)),
                  _full((1, F))],
        out_specs=pl.BlockSpec((BE, F), lambda i: (i, 0)),
        out_shape=jax.ShapeDtypeStruct((E_PAD, F), f32),
    )(hsrc, hdst, d_p, w1, w2, w3, b2)

    aggsrc = None
    for b in range(N_BLOCKS):
        if b == 0:
            mm = pl.pallas_call(
                _msg_first_body,
                grid=(E_PAD // BE,),
                in_specs=[pl.BlockSpec((BE, F), lambda i: (i, 0)),
                          pl.BlockSpec((BE, 1), lambda i: (i, 0)),
                          _full((RB, F)), _full((F, F))],
                out_specs=pl.BlockSpec((BE, F), lambda i: (i, 0)),
                out_shape=jax.ShapeDtypeStruct((E_PAD, F), f32),
            )(m, d_p, w_rbf[b], w_msg[b])
        else:
            mm, m = pl.pallas_call(
                _msg_body,
                grid=(E_PAD // BE,),
                in_specs=[pl.BlockSpec((BE, F), lambda i: (i, 0)),
                          pl.BlockSpec((BE, F), lambda i: (i, 0)),
                          pl.BlockSpec((BE, 1), lambda i: (i, 0)),
                          _full((RB, F)), _full((F, F))],
                out_specs=[pl.BlockSpec((BE, F), lambda i: (i, 0)),
                           pl.BlockSpec((BE, F), lambda i: (i, 0))],
                out_shape=[jax.ShapeDtypeStruct((E_PAD, F), f32),
                           jax.ShapeDtypeStruct((E_PAD, F), f32)],
            )(m, aggsrc, d_p, w_rbf[b], w_msg[b])

        parts = _sc_segment_sum(mm, dst3, zeros_f)
        aggsrc = _sc_gather([parts[0], parts[1]], src_p)

    m4 = pl.pallas_call(
        _add_body,
        grid=(E_PAD // BE,),
        in_specs=[pl.BlockSpec((BE, F), lambda i: (i, 0)),
                  pl.BlockSpec((BE, F), lambda i: (i, 0))],
        out_specs=pl.BlockSpec((BE, F), lambda i: (i, 0)),
        out_shape=jax.ShapeDtypeStruct((E_PAD, F), f32),
    )(m, aggsrc)

    parts4 = _sc_segment_sum(m4, dst3, zeros_f)

    out = pl.pallas_call(
        _final_body,
        grid=(N_PAD // BA,),
        in_specs=[pl.BlockSpec((BA, F), lambda i: (i, 0)),
                  pl.BlockSpec((BA, F), lambda i: (i, 0)),
                  _full((F, 16)), _full((1, 1))],
        out_specs=pl.BlockSpec((BA, 1), lambda i: (i, 0)),
        out_shape=jax.ShapeDtypeStruct((N_PAD, 1), f32),
    )(parts4[0], parts4[1], w16, b11)

    return out[:n_atoms, 0]


# speedup vs baseline: 1.5329x; 1.5329x over previous
import functools

import jax
import jax.numpy as jnp
from jax import lax
from jax.experimental import pallas as pl
from jax.experimental.pallas import tpu as pltpu
from jax.experimental.pallas import tpu_sc as plsc

F = 128
RB = 16
CUTOFF = 0.5
N_BLOCKS = 4

E_PAD = 163840
N_PAD = 10240
BE = 2048
BA = 2048
GW = 256
CH = 320

_VMESH = plsc.VectorSubcoreMesh(core_axis_name="c", subcore_axis_name="s")


def _h_body(z_ref, emb_ref, h_ref):
    z = z_ref[...]
    cols = lax.broadcasted_iota(jnp.int32, (BA, F), 1)
    onehot = (cols == z).astype(jnp.float32)
    h_ref[...] = jnp.dot(onehot, emb_ref[...],
                         preferred_element_type=jnp.float32)


_A1 = 6.283185307179586
_A3 = -41.341702240399755
_A5 = 81.60524927607504
_A7 = -76.70585975306136
_A9 = 42.05869394489765
_A11 = -15.094642576822022


def _sin_turns(t):
    r = t - jnp.round(t)
    fold = jnp.where(r > 0, 0.5 - r, -0.5 - r)
    r = jnp.where(jnp.abs(r) > 0.25, fold, r)
    z = r * r
    p = _A9 + z * _A11
    p = _A7 + z * p
    p = _A5 + z * p
    p = _A3 + z * p
    p = _A1 + z * p
    return r * p


def _edge_basis(d_ref):
    dd = d_ref[...] + 1e-3
    j = lax.broadcasted_iota(jnp.int32, (BE, 32), 1)
    jf = (j + 1).astype(jnp.float32)
    n = jnp.where(j < 16, jf, jnp.where(j == 16, 1.0, 0.0))
    shf = jnp.where(j == 16, 0.25, 0.0)
    s = _sin_turns(dd * n + shf)
    sin16 = s[:, :RB]
    c = s[:, RB:RB + 1]
    fcut = 0.5 * (c + 1.0) * (dd < CUTOFF).astype(jnp.float32)
    rowscale = 2.0 * fcut / dd
    return sin16, rowscale


def _silu(x):
    return x * jax.nn.sigmoid(x)


def _embed_body(hs_ref, hd_ref, d_ref, w1_ref, w2_ref, w3_ref, b_ref, m_ref):
    sin16, rowscale = _edge_basis(d_ref)
    x = (jnp.dot(hs_ref[...], w1_ref[...], preferred_element_type=jnp.float32)
         + jnp.dot(hd_ref[...], w2_ref[...], preferred_element_type=jnp.float32)
         + jnp.dot(sin16, w3_ref[...],
                   preferred_element_type=jnp.float32) * rowscale
         + b_ref[...])
    m_ref[...] = _silu(x)


def _msg_first_body(m_ref, d_ref, wr_ref, wm_ref, mm_ref):
    sin16, rowscale = _edge_basis(d_ref)
    g = jnp.dot(sin16, wr_ref[...],
                preferred_element_type=jnp.float32) * rowscale
    x = jnp.dot(m_ref[...] * g, wm_ref[...], preferred_element_type=jnp.float32)
    mm_ref[...] = _silu(x)


def _msg_body(m_ref, agg_ref, d_ref, wr_ref, wm_ref, mm_ref, mnew_ref):
    m = m_ref[...] + agg_ref[...]
    sin16, rowscale = _edge_basis(d_ref)
    g = jnp.dot(sin16, wr_ref[...],
                preferred_element_type=jnp.float32) * rowscale
    x = jnp.dot(m * g, wm_ref[...], preferred_element_type=jnp.float32)
    mm_ref[...] = _silu(x)
    mnew_ref[...] = m


def _add_body(m_ref, agg_ref, o_ref):
    o_ref[...] = m_ref[...] + agg_ref[...]


def _final_body(p0_ref, p1_ref, w16_ref, b_ref, o_ref):
    s = p0_ref[...] + p1_ref[...]
    e = jnp.dot(s, w16_ref[...], preferred_element_type=jnp.float32)
    o_ref[...] = e[:, 0:1] + b_ref[...]


def _full(shape):
    return pl.BlockSpec(shape, lambda i: tuple(0 for _ in shape))


def _sc_gather(tables, idx):
    e = idx.shape[0]
    width = tables[0].shape[1]
    idx2 = idx.reshape(1, e)

    @functools.partial(
        pl.kernel,
        out_type=jax.ShapeDtypeStruct((e, width), tables[0].dtype),
        mesh=_VMESH,
    )
    def k(*refs):
        tab_refs = refs[:len(tables)]
        i_hbm = refs[len(tables)]
        o_hbm = refs[len(tables) + 1]

        def body(i_vmem, o_vmem):
            pltpu.sync_copy(tab_refs[0].at[i_vmem.at[0]], o_vmem)
            for t in tab_refs[1:]:
                pltpu.sync_copy(t.at[i_vmem.at[0]], o_vmem, add=True)

        pltpu.emit_pipeline(
            body,
            grid=(e // GW,),
            in_specs=[pl.BlockSpec((1, GW), lambda i: (0, i))],
            out_specs=[pl.BlockSpec((GW, width), lambda i: (i, 0))],
            core_axis_name=("c", "s"),
            dimension_semantics=(pltpu.PARALLEL,),
        )(i_hbm, o_hbm)

    return k(*tables, idx2)


def _sc_segment_sum(vals, dst3, zeros):
    width = vals.shape[1]
    nch_w = E_PAD // CH // 32
    rs = N_PAD // 16

    @functools.partial(
        pl.kernel,
        out_type=jax.ShapeDtypeStruct((2, N_PAD, width), jnp.float32),
        mesh=_VMESH,
        scratch_types=[
            pltpu.VMEM_SHARED((N_PAD, width), jnp.float32),
            pltpu.VMEM((CH, width), jnp.float32),
            pltpu.VMEM((1, CH), jnp.int32),
        ],
    )
    def k(vals_hbm, dst_hbm, z_hbm, out_hbm, agg_sh, rows_v, idx_v):
        cid = lax.axis_index("c")
        sid = lax.axis_index("s")
        pltpu.sync_copy(z_hbm.at[pl.ds(sid * rs, rs)],
                        agg_sh.at[pl.ds(sid * rs, rs)])
        plsc.subcore_barrier()

        w = cid * 16 + sid

        @pl.loop(0, nch_w)
        def _(j):
            ch = w * nch_w + j
            pltpu.sync_copy(dst_hbm.at[ch], idx_v)
            pltpu.sync_copy(vals_hbm.at[pl.ds(ch * CH, CH)], rows_v)
            pltpu.sync_copy(rows_v, agg_sh.at[idx_v.at[0]], add=True)

        plsc.subcore_barrier()
        pltpu.sync_copy(agg_sh.at[pl.ds(sid * rs, rs)],
                        out_hbm.at[cid].at[pl.ds(sid * rs, rs)])

    return k(vals, dst3, zeros)


def kernel(atomic_numbers, positions, pair_indices, d_ij,
           atomic_subsystem_indices, emb_table, w_embed, b_embed,
           w_rbf, w_msg, w_out, b_out):
    n_atoms = atomic_numbers.shape[0]
    e = pair_indices.shape[1]
    f32 = jnp.float32

    pad_a = N_PAD - n_atoms
    pad_e = E_PAD - e
    z_p = jnp.concatenate(
        [atomic_numbers.astype(jnp.int32),
         jnp.full((pad_a,), 101, jnp.int32)]).reshape(N_PAD, 1)
    emb_pad = jnp.zeros((F, F), f32).at[:emb_table.shape[0]].set(emb_table)
    src_p = jnp.concatenate([pair_indices[0].astype(jnp.int32),
                             jnp.full((pad_e,), n_atoms, jnp.int32)])
    dst_p = jnp.concatenate([pair_indices[1].astype(jnp.int32),
                             jnp.full((pad_e,), n_atoms, jnp.int32)])
    dst3 = dst_p.reshape(E_PAD // CH, 1, CH)
    d_p = jnp.concatenate([d_ij.astype(f32),
                           jnp.zeros((pad_e, 1), f32)])
    w1 = w_embed[:F]
    w2 = w_embed[F:2 * F]
    w3 = w_embed[2 * F:]
    b2 = b_embed.reshape(1, F)
    w16 = jnp.tile(w_out, (1, 16))
    b11 = b_out.reshape(1, 1)
    zeros_f = jnp.zeros((N_PAD, F), f32)

    h = pl.pallas_call(
        _h_body,
        grid=(N_PAD // BA,),
        in_specs=[pl.BlockSpec((BA, 1), lambda i: (i, 0)), _full((F, F))],
        out_specs=pl.BlockSpec((BA, F), lambda i: (i, 0)),
        out_shape=jax.ShapeDtypeStruct((N_PAD, F), f32),
    )(z_p, emb_pad)

    hsrc = _sc_gather([h], src_p)
    hdst = _sc_gather([h], dst_p)

    m = pl.pallas_call(
        _embed_body,
        grid=(E_PAD // BE,),
        in_specs=[pl.BlockSpec((BE, F), lambda i: (i, 0)),
                  pl.BlockSpec((BE, F), lambda i: (i, 0)),
                  pl.BlockSpec((BE, 1), lambda i: (i, 0)),
                  _full((F, F)), _full((F, F)), _full((RB, F)),
                  _full((1, F))],
        out_specs=pl.BlockSpec((BE, F), lambda i: (i, 0)),
        out_shape=jax.ShapeDtypeStruct((E_PAD, F), f32),
    )(hsrc, hdst, d_p, w1, w2, w3, b2)

    aggsrc = None
    for b in range(N_BLOCKS):
        if b == 0:
            mm = pl.pallas_call(
                _msg_first_body,
                grid=(E_PAD // BE,),
                in_specs=[pl.BlockSpec((BE, F), lambda i: (i, 0)),
                          pl.BlockSpec((BE, 1), lambda i: (i, 0)),
                          _full((RB, F)), _full((F, F))],
                out_specs=pl.BlockSpec((BE, F), lambda i: (i, 0)),
                out_shape=jax.ShapeDtypeStruct((E_PAD, F), f32),
            )(m, d_p, w_rbf[b], w_msg[b])
        else:
            mm, m = pl.pallas_call(
                _msg_body,
                grid=(E_PAD // BE,),
                in_specs=[pl.BlockSpec((BE, F), lambda i: (i, 0)),
                          pl.BlockSpec((BE, F), lambda i: (i, 0)),
                          pl.BlockSpec((BE, 1), lambda i: (i, 0)),
                          _full((RB, F)), _full((F, F))],
                out_specs=[pl.BlockSpec((BE, F), lambda i: (i, 0)),
                           pl.BlockSpec((BE, F), lambda i: (i, 0))],
                out_shape=[jax.ShapeDtypeStruct((E_PAD, F), f32),
                           jax.ShapeDtypeStruct((E_PAD, F), f32)],
            )(m, aggsrc, d_p, w_rbf[b], w_msg[b])

        parts = _sc_segment_sum(mm, dst3, zeros_f)
        aggsrc = _sc_gather([parts[0], parts[1]], src_p)

    m4 = pl.pallas_call(
        _add_body,
        grid=(E_PAD // BE,),
        in_specs=[pl.BlockSpec((BE, F), lambda i: (i, 0)),
                  pl.BlockSpec((BE, F), lambda i: (i, 0))],
        out_specs=pl.BlockSpec((BE, F), lambda i: (i, 0)),
        out_shape=jax.ShapeDtypeStruct((E_PAD, F), f32),
    )(m, aggsrc)

    parts4 = _sc_segment_sum(m4, dst3, zeros_f)

    out = pl.pallas_call(
        _final_body,
        grid=(N_PAD // BA,),
        in_specs=[pl.BlockSpec((BA, F), lambda i: (i, 0)),
                  pl.BlockSpec((BA, F), lambda i: (i, 0)),
                  _full((F, 16)), _full((1, 1))],
        out_specs=pl.BlockSpec((BA, 1), lambda i: (i, 0)),
        out_shape=jax.ShapeDtypeStruct((N_PAD, 1), f32),
    )(parts4[0], parts4[1], w16, b11)

    return out[:n_atoms, 0]
